# baseline (speedup 1.0000x reference)
import functools

import jax
import jax.numpy as jnp
from jax import lax
from jax.experimental import pallas as pl
from jax.experimental.pallas import tpu as pltpu
from jax.experimental.pallas import tpu_sc as plsc

_SC_NC = 2
_SC_NS = 16
_SC_NW = _SC_NC * _SC_NS


_SEGS = ((0, 2816), (2816, 5632), (5632, 8192))

def _argmin_body(z_ref, emb_ref, idx_ref, e2_ref, iota_ref):
    i = pl.program_id(0)

    @pl.when(i == 0)
    def _():
        e = emb_ref[...]
        e2_ref[...] = jnp.sum(e * e, axis=0, keepdims=True)
        iota_ref[...] = lax.broadcasted_iota(
            jnp.int32, iota_ref.shape, 1).astype(jnp.float32)

    d = z_ref.shape[1]
    z = jnp.transpose(z_ref[...].reshape(d, -1), (1, 0))
    s = jnp.dot(z.astype(jnp.bfloat16), emb_ref[...].astype(jnp.bfloat16),
                preferred_element_type=jnp.float32)
    z2 = jnp.sum(z * z, axis=1, keepdims=True)
    dist = (z2 - 2.0 * s) + e2_ref[...]
    iota = iota_ref[...]
    av = None
    for lo, hi in _SEGS:
        seg = dist[:, lo:hi]
        m = jnp.min(seg, axis=1, keepdims=True)
        ix = jnp.min(jnp.where(seg == m, iota[:, lo:hi], jnp.float32(2e9)),
                     axis=1, keepdims=True)
        if av is None:
            av = m.astype(jnp.bfloat16).astype(jnp.float32)
            ai = ix
        else:
            keep = (av < m) | ((av == m) & (ai < ix))
            av = jnp.where(keep, av, m).astype(jnp.bfloat16).astype(jnp.float32)
            ai = jnp.where(keep, ai, ix)
    idx_ref[...] = ai.astype(jnp.int32).reshape(1, 1, ai.shape[0])


def _tc_argmin(z_e, emb):
    b, d, h, w = z_e.shape
    row_block = h * w
    n = b * row_block
    k = emb.shape[1]
    idx3 = pl.pallas_call(
        _argmin_body,
        grid=(b,),
        in_specs=[
            pl.BlockSpec((1, d, h, w), lambda i: (i, 0, 0, 0)),
            pl.BlockSpec((d, k), lambda i: (0, 0)),
        ],
        out_specs=pl.BlockSpec((1, 1, row_block), lambda i: (i, 0, 0)),
        out_shape=jax.ShapeDtypeStruct((b, 1, row_block), jnp.int32),
        scratch_shapes=[pltpu.VMEM((1, k), jnp.float32),
                        pltpu.VMEM((1, k), jnp.float32)],
    )(z_e, emb)
    return idx3.reshape(n)


def _sc_gather(table, idx):
    n, d = idx.shape[0], table.shape[1]
    b_per_w = n // _SC_NW
    chunk = 128
    n_chunks = b_per_w // chunk
    mesh = plsc.VectorSubcoreMesh(core_axis_name="c", subcore_axis_name="s")

    @functools.partial(
        pl.kernel,
        mesh=mesh,
        out_type=jax.ShapeDtypeStruct((n, d), jnp.float32),
        scratch_types=[
            pltpu.VMEM((n_chunks, chunk), jnp.int32),
            pltpu.VMEM((b_per_w, d), jnp.float32),
            pltpu.SemaphoreType.DMA,
        ],
    )
    def gather_kernel(table_hbm, idx_hbm, out_hbm, idx_v, rows_v, sem):
        wid = lax.axis_index("s") * _SC_NC + lax.axis_index("c")
        base = wid * b_per_w
        for g in range(n_chunks):
            pltpu.sync_copy(
                idx_hbm.at[pl.ds(base + g * chunk, chunk)],
                idx_v.at[g],
            )
        copies = []
        for g in range(n_chunks):
            copies.append(pltpu.async_copy(
                table_hbm.at[idx_v.at[g]],
                rows_v.at[pl.ds(g * chunk, chunk)],
                sem,
            ))
        for c in copies:
            c.wait()
        pltpu.sync_copy(rows_v, out_hbm.at[pl.ds(base, b_per_w)])

    return gather_kernel(table, idx)


def kernel(z_e, emb):
    b, d, h, w = z_e.shape
    n = b * h * w
    idx = _tc_argmin(z_e, emb)
    zq_flat = _sc_gather(emb.T, idx)
    z_q = jnp.transpose(zq_flat.reshape(b, h, w, d), (0, 3, 1, 2))
    return z_q, idx.reshape(b, h, w)


# speedup vs baseline: 1.1282x; 1.1282x over previous
import functools

import jax
import jax.numpy as jnp
from jax import lax
from jax.experimental import pallas as pl
from jax.experimental.pallas import tpu as pltpu
from jax.experimental.pallas import tpu_sc as plsc

_SC_NC = 2
_SC_NS = 16
_SC_NW = _SC_NC * _SC_NS


_SEGS = ((0, 2816), (2816, 5632), (5632, 8192))

def _argmin_body(z_ref, emb_ref, idx_ref, e2_ref, iota_ref):
    i = pl.program_id(0)

    @pl.when(i == 0)
    def _():
        e = emb_ref[...]
        e2_ref[...] = jnp.sum(e * e, axis=0, keepdims=True)
        iota_ref[...] = lax.broadcasted_iota(
            jnp.int32, iota_ref.shape, 1).astype(jnp.float32)

    z = z_ref[...]
    s = jnp.dot(z.astype(jnp.bfloat16), emb_ref[...].astype(jnp.bfloat16),
                preferred_element_type=jnp.float32)
    z2 = jnp.sum(z * z, axis=1, keepdims=True)
    dist = (z2 - 2.0 * s) + e2_ref[...]
    iota = iota_ref[...]
    av = None
    for lo, hi in _SEGS:
        seg = dist[:, lo:hi]
        m = jnp.min(seg, axis=1, keepdims=True)
        ix = jnp.min(jnp.where(seg == m, iota[:, lo:hi], jnp.float32(2e9)),
                     axis=1, keepdims=True)
        if av is None:
            av = m.astype(jnp.bfloat16).astype(jnp.float32)
            ai = ix
        else:
            keep = (av < m) | ((av == m) & (ai < ix))
            av = jnp.where(keep, av, m).astype(jnp.bfloat16).astype(jnp.float32)
            ai = jnp.where(keep, ai, ix)
    idx_ref[...] = ai.astype(jnp.int32).reshape(1, 1, ai.shape[0])


def _tc_argmin(z, emb, row_block):
    n, d = z.shape
    k = emb.shape[1]
    nb = n // row_block
    idx3 = pl.pallas_call(
        _argmin_body,
        grid=(nb,),
        in_specs=[
            pl.BlockSpec((row_block, d), lambda i: (i, 0)),
            pl.BlockSpec((d, k), lambda i: (0, 0)),
        ],
        out_specs=pl.BlockSpec((1, 1, row_block), lambda i: (i, 0, 0)),
        out_shape=jax.ShapeDtypeStruct((nb, 1, row_block), jnp.int32),
        scratch_shapes=[pltpu.VMEM((1, k), jnp.float32),
                        pltpu.VMEM((1, k), jnp.float32)],
    )(z, emb)
    return idx3.reshape(n)


def _sc_gather(table, idx):
    n, d = idx.shape[0], table.shape[1]
    b_per_w = n // _SC_NW
    chunk = 128
    n_chunks = b_per_w // chunk
    mesh = plsc.VectorSubcoreMesh(core_axis_name="c", subcore_axis_name="s")

    @functools.partial(
        pl.kernel,
        mesh=mesh,
        out_type=jax.ShapeDtypeStruct((n, d), jnp.float32),
        scratch_types=[
            pltpu.VMEM((n_chunks, chunk), jnp.int32),
            pltpu.VMEM((b_per_w, d), jnp.float32),
            pltpu.SemaphoreType.DMA,
        ],
    )
    def gather_kernel(table_hbm, idx_hbm, out_hbm, idx_v, rows_v, sem):
        wid = lax.axis_index("s") * _SC_NC + lax.axis_index("c")
        base = wid * b_per_w
        for g in range(n_chunks):
            pltpu.sync_copy(
                idx_hbm.at[pl.ds(base + g * chunk, chunk)],
                idx_v.at[g],
            )
        copies = []
        for g in range(n_chunks):
            copies.append(pltpu.async_copy(
                table_hbm.at[idx_v.at[g]],
                rows_v.at[pl.ds(g * chunk, chunk)],
                sem,
            ))
        for c in copies:
            c.wait()
        pltpu.sync_copy(rows_v, out_hbm.at[pl.ds(base, b_per_w)])

    return gather_kernel(table, idx)


def kernel(z_e, emb):
    b, d, h, w = z_e.shape
    n = b * h * w
    z = jnp.transpose(z_e, (0, 2, 3, 1)).reshape(n, d)
    emb_t = emb.T
    idx1 = _tc_argmin(z[: n // 2], emb, row_block=1024)
    zq1 = _sc_gather(emb_t, idx1)
    idx2 = _tc_argmin(z[n // 2:], emb, row_block=1024)
    zq2 = _sc_gather(emb_t, idx2)
    idx = jnp.concatenate([idx1, idx2])
    zq_flat = jnp.concatenate([zq1, zq2], axis=0)
    z_q = jnp.transpose(zq_flat.reshape(b, h, w, d), (0, 3, 1, 2))
    return z_q, idx.reshape(b, h, w)


# speedup vs baseline: 1.3907x; 1.2326x over previous
import functools

import jax
import jax.numpy as jnp
from jax import lax
from jax.experimental import pallas as pl
from jax.experimental.pallas import tpu as pltpu
from jax.experimental.pallas import tpu_sc as plsc

_SC_NC = 2
_SC_NS = 16
_SC_NW = _SC_NC * _SC_NS


_SEGS = ((0, 2816), (2816, 5632), (5632, 8192))

def _argmin_body(z_ref, emb_ref, idx_ref, e2_ref, iota_ref):
    i = pl.program_id(0)

    @pl.when(i == 0)
    def _():
        e = emb_ref[...]
        e2_ref[...] = jnp.sum(e * e, axis=0, keepdims=True)
        iota_ref[...] = lax.broadcasted_iota(
            jnp.int32, iota_ref.shape, 1).astype(jnp.float32)

    z = z_ref[...]
    s = jnp.dot(z.astype(jnp.bfloat16), emb_ref[...].astype(jnp.bfloat16),
                preferred_element_type=jnp.float32)
    z2 = jnp.sum(z * z, axis=1, keepdims=True)
    dist = (z2 - 2.0 * s) + e2_ref[...]
    iota = iota_ref[...]
    av = None
    for lo, hi in _SEGS:
        seg = dist[:, lo:hi]
        m = jnp.min(seg, axis=1, keepdims=True)
        ix = jnp.min(jnp.where(seg == m, iota[:, lo:hi], jnp.float32(2e9)),
                     axis=1, keepdims=True)
        if av is None:
            av = m.astype(jnp.bfloat16).astype(jnp.float32)
            ai = ix
        else:
            keep = (av < m) | ((av == m) & (ai < ix))
            av = jnp.where(keep, av, m).astype(jnp.bfloat16).astype(jnp.float32)
            ai = jnp.where(keep, ai, ix)
    idx_ref[...] = ai.astype(jnp.int32).reshape(1, 1, ai.shape[0])


def _tc_argmin(z, emb, row_block):
    n, d = z.shape
    k = emb.shape[1]
    nb = n // row_block
    idx3 = pl.pallas_call(
        _argmin_body,
        grid=(nb,),
        in_specs=[
            pl.BlockSpec((row_block, d), lambda i: (i, 0)),
            pl.BlockSpec((d, k), lambda i: (0, 0)),
        ],
        out_specs=pl.BlockSpec((1, 1, row_block), lambda i: (i, 0, 0)),
        out_shape=jax.ShapeDtypeStruct((nb, 1, row_block), jnp.int32),
        scratch_shapes=[pltpu.VMEM((1, k), jnp.float32),
                        pltpu.VMEM((1, k), jnp.float32)],
    )(z, emb)
    return idx3.reshape(n)


def _sc_gather(table, idx):
    n, d = idx.shape[0], table.shape[1]
    b_per_w = n // _SC_NW
    chunk = 128
    n_chunks = b_per_w // chunk
    mesh = plsc.VectorSubcoreMesh(core_axis_name="c", subcore_axis_name="s")

    @functools.partial(
        pl.kernel,
        mesh=mesh,
        out_type=jax.ShapeDtypeStruct((n, d), jnp.float32),
        scratch_types=[
            pltpu.VMEM((n_chunks, chunk), jnp.int32),
            pltpu.VMEM((b_per_w, d), jnp.float32),
            pltpu.SemaphoreType.DMA,
        ],
    )
    def gather_kernel(table_hbm, idx_hbm, out_hbm, idx_v, rows_v, sem):
        wid = lax.axis_index("s") * _SC_NC + lax.axis_index("c")
        base = wid * b_per_w
        for g in range(n_chunks):
            pltpu.sync_copy(
                idx_hbm.at[pl.ds(base + g * chunk, chunk)],
                idx_v.at[g],
            )
        copies = []
        for g in range(n_chunks):
            copies.append(pltpu.async_copy(
                table_hbm.at[idx_v.at[g]],
                rows_v.at[pl.ds(g * chunk, chunk)],
                sem,
            ))
        for c in copies:
            c.wait()
        pltpu.sync_copy(rows_v, out_hbm.at[pl.ds(base, b_per_w)])

    return gather_kernel(table, idx)


def kernel(z_e, emb):
    b, d, h, w = z_e.shape
    n = b * h * w
    z = jnp.transpose(z_e, (0, 2, 3, 1)).reshape(n, d)
    idx = _tc_argmin(z, emb, row_block=1024)
    zq_flat = _sc_gather(emb.T, idx)
    z_q = jnp.transpose(zq_flat.reshape(b, h, w, d), (0, 3, 1, 2))
    return z_q, idx.reshape(b, h, w)
